# baseline (speedup 1.0000x reference)
import dataclasses
import functools

import jax
import jax.numpy as jnp
from jax import lax
from jax.experimental import pallas as pl
from jax.experimental.pallas import tpu as pltpu
from jax.experimental.pallas import tpu_sc as plsc

N = 10000
E = 320000
BAG = 16
D = 128
C = 16

NC = 2
NS = 16
NW = NC * NS
L = 16

EMB_BLOCKS = (N * BAG) // 128
EMB_NB = 40
EMB_ROWS = 10240
EDGE_BLOCKS = E // 128
ROW_BLOCKS = N // 8

CB = 160
CROWS = 64
PAD_BLOCKS = CB * NS * NC
E_PAD = PAD_BLOCKS * CROWS
NB = 80
ZB = 16
ACC_ROWS = 10240

_mesh = plsc.VectorSubcoreMesh(core_axis_name="c", subcore_axis_name="s")

_sc_params = pltpu.CompilerParams()
if "needs_layout_passes" in pltpu.CompilerParams.__dataclass_fields__:
    _sc_params = dataclasses.replace(_sc_params, needs_layout_passes=False)


def _ceil_div(a, b):
    return (a + b - 1) // b


@functools.partial(
    pl.kernel,
    out_type=(
        jax.ShapeDtypeStruct((EMB_ROWS, D), jnp.float32),
        jax.ShapeDtypeStruct((NW, N), jnp.float32),
    ),
    mesh=_mesh,
    scratch_types=[
        pltpu.VMEM((EMB_NB, 128), jnp.int32),
        pltpu.VMEM((CB // 2, CROWS), jnp.int32),
        pltpu.VMEM((128, D), jnp.float32),
        pltpu.VMEM((128, D), jnp.float32),
        pltpu.VMEM((8, D), jnp.float32),
        pltpu.VMEM((8, D), jnp.float32),
        pltpu.VMEM((N,), jnp.float32),
        pltpu.SemaphoreType.DMA,
        pltpu.SemaphoreType.DMA,
        pltpu.SemaphoreType.DMA,
        pltpu.SemaphoreType.DMA,
    ],
    compiler_params=_sc_params,
)
def _sc_emb_deg(x_hbm, eidx_hbm, table_hbm, hsum_hbm, degp_hbm,
                idx_v, didx_v, rows0, rows1, acc0, acc1, hist_v,
                gsem0, gsem1, wsem0, wsem1):
    c = lax.axis_index("c")
    s = lax.axis_index("s")
    wid = s * NC + c

    ones = jnp.ones((L,), jnp.float32)
    zeros = jnp.zeros((L,), jnp.float32)

    @pl.loop(0, N // L)
    def _(k):
        hist_v[pl.ds(k * L, L)] = zeros

    lo_e = PAD_BLOCKS + wid * CB
    for h in range(2):
        pltpu.sync_copy(eidx_hbm.at[pl.ds(lo_e + h * (CB // 2), CB // 2)],
                        didx_v)

        @pl.loop(0, CB // 2)
        def _(b):
            for j in range(CROWS // L):
                idx = didx_v.at[pl.ds(b, 1)][0, pl.ds(j * L, L)]
                plsc.addupdate_scatter(hist_v, [idx], ones, mask=idx < N)

    pltpu.sync_copy(hist_v, degp_hbm.at[wid])

    lo_x = wid * EMB_NB
    pltpu.sync_copy(x_hbm.at[pl.ds(lo_x, EMB_NB)], idx_v.at[pl.ds(0, EMB_NB)])

    def gather(blk, buf, sem):
        pltpu.async_copy(table_hbm.at[idx_v.at[blk]], buf, sem)

    def wait_gather(buf, sem):
        pltpu.make_async_copy(table_hbm.at[idx_v.at[0]], buf, sem).wait()

    def wait_write(buf, sem):
        pltpu.make_async_copy(buf, hsum_hbm.at[pl.ds(0, 8)], sem).wait()

    def reduce_block(rbuf, abuf):
        for n in range(8):
            for j in range(D // L):
                sl = pl.ds(j * L, L)
                v = rbuf.at[pl.ds(n * BAG, 1)][0, sl]
                for t in range(1, BAG):
                    v = v + rbuf.at[pl.ds(n * BAG + t, 1)][0, sl]
                abuf.at[pl.ds(n, 1)][0, sl] = v

    gather(0, rows0, gsem0)

    @pl.loop(0, EMB_NB // 2)
    def _(i):
        e = 2 * i
        o = e + 1
        wait_gather(rows0, gsem0)
        gather(o, rows1, gsem1)

        @pl.when(i > 0)
        def _():
            wait_write(acc0, wsem0)

        reduce_block(rows0, acc0)
        pltpu.async_copy(acc0, hsum_hbm.at[pl.ds((lo_x + e) * 8, 8)], wsem0)
        wait_gather(rows1, gsem1)

        @pl.when(i < EMB_NB // 2 - 1)
        def _():
            gather(e + 2, rows0, gsem0)

        @pl.when(i > 0)
        def _():
            wait_write(acc1, wsem1)

        reduce_block(rows1, acc1)
        pltpu.async_copy(acc1, hsum_hbm.at[pl.ds((lo_x + o) * 8, 8)], wsem1)

    wait_write(acc0, wsem0)
    wait_write(acc1, wsem1)


@functools.partial(
    pl.kernel,
    out_type=jax.ShapeDtypeStruct((NC, ACC_ROWS, D), jnp.float32),
    mesh=_mesh,
    scratch_types=[
        pltpu.VMEM((CB // 4, CROWS), jnp.int32),
        pltpu.VMEM((CB // 4, CROWS), jnp.int32),
        pltpu.VMEM((4, CROWS, D), jnp.float32),
        pltpu.VMEM((ZB, D), jnp.float32),
        pltpu.VMEM_SHARED((ACC_ROWS, D), jnp.float32),
        [pltpu.SemaphoreType.DMA] * 4,
        [pltpu.SemaphoreType.DMA] * 4,
        pltpu.SemaphoreType.DMA,
    ],
    compiler_params=_sc_params,
)
def _sc_scatter(g_hbm, eidx_hbm, out_hbm,
                sidx_v, didx_v, rows_v, zero_v, acc_sh,
                gsems, ssems, osem):
    c = lax.axis_index("c")
    s = lax.axis_index("s")

    zeros = jnp.zeros((L,), jnp.float32)

    @pl.loop(0, ZB)
    def _(r):
        for j in range(D // L):
            zero_v.at[pl.ds(r, 1)][0, pl.ds(j * L, L)] = zeros

    cpt = ACC_ROWS // ZB // NS
    for i in range(cpt):
        b = s * cpt + i
        pltpu.async_copy(zero_v, acc_sh.at[pl.ds(b * ZB, ZB)], osem)
    for i in range(cpt):
        pltpu.make_async_copy(zero_v, acc_sh.at[pl.ds(0, ZB)], osem).wait()

    plsc.subcore_barrier()

    bufs = [rows_v.at[k] for k in range(4)]

    def gather(blk, k):
        pltpu.async_copy(g_hbm.at[sidx_v.at[blk]], bufs[k], gsems[k])

    def wait_gather(k):
        pltpu.make_async_copy(g_hbm.at[sidx_v.at[0]], bufs[k],
                              gsems[k]).wait()

    def scatter(blk, k):
        pltpu.async_copy(bufs[k], acc_sh.at[didx_v.at[blk]], ssems[k],
                         add=True)

    def wait_scatter(k):
        pltpu.make_async_copy(bufs[k], acc_sh.at[didx_v.at[0]],
                              ssems[k]).wait()

    CB2 = CB // 4
    lo = c * (CB * NS) + s * CB
    for h in range(4):
        pltpu.sync_copy(eidx_hbm.at[pl.ds(lo + h * CB2, CB2)], sidx_v)
        pltpu.sync_copy(eidx_hbm.at[pl.ds(PAD_BLOCKS + lo + h * CB2, CB2)],
                        didx_v)
        gather(0, 0)
        gather(2, 2)

        @pl.loop(0, CB2 // 4)
        def _(i):
            a = 4 * i
            wait_gather(0)
            scatter(a, 0)

            @pl.when(i > 0)
            def _():
                wait_scatter(1)

            gather(a + 1, 1)
            wait_gather(2)
            scatter(a + 2, 2)

            @pl.when(i > 0)
            def _():
                wait_scatter(3)

            gather(a + 3, 3)
            wait_gather(1)
            scatter(a + 1, 1)

            @pl.when(i < CB2 // 4 - 1)
            def _():
                wait_scatter(0)
                gather(a + 4, 0)

            wait_gather(3)
            scatter(a + 3, 3)

            @pl.when(i < CB2 // 4 - 1)
            def _():
                wait_scatter(2)
                gather(a + 6, 2)

        for k in range(4):
            wait_scatter(k)

    plsc.subcore_barrier()

    for i in range(cpt):
        b = s * cpt + i
        pltpu.async_copy(acc_sh.at[pl.ds(b * ZB, ZB)],
                         out_hbm.at[c, pl.ds(b * ZB, ZB)], osem)
    for i in range(cpt):
        pltpu.make_async_copy(acc_sh.at[pl.ds(0, ZB)],
                              out_hbm.at[c, pl.ds(0, ZB)], osem).wait()


_BM = 2000


def _dinv_of(degp_block):
    deg = jnp.sum(degp_block, axis=1, keepdims=True) + 1.0
    return lax.rsqrt(deg)


def _tc1_body(hs_ref, degp_ref, w_ref, g_ref):
    h = jnp.maximum(hs_ref[...] * (1.0 / BAG), 0.0)
    hw = jnp.dot(h, w_ref[...], preferred_element_type=jnp.float32)
    g_ref[...] = hw * _dinv_of(degp_ref[...])


def _tc1(h_sum, degp, W1):
    return pl.pallas_call(
        _tc1_body,
        grid=(N // _BM,),
        in_specs=[
            pl.BlockSpec((_BM, D), lambda i: (i, 0)),
            pl.BlockSpec((_BM, NW), lambda i: (i, 0)),
            pl.BlockSpec((D, D), lambda i: (0, 0)),
        ],
        out_specs=pl.BlockSpec((_BM, D), lambda i: (i, 0)),
        out_shape=jax.ShapeDtypeStruct((N, D), jnp.float32),
    )(h_sum, degp, W1)


def _tc2_body(p_ref, g_ref, degp_ref, w_ref, out_ref):
    dinv = _dinv_of(degp_ref[...])
    ssum = p_ref[0] + p_ref[1] + g_ref[...]
    out1 = jnp.maximum(ssum * dinv, 0.0)
    hw = jnp.dot(out1, w_ref[...], preferred_element_type=jnp.float32)
    out_ref[...] = hw * dinv


def _tc2(p, g1, degp, W2):
    return pl.pallas_call(
        _tc2_body,
        grid=(N // _BM,),
        in_specs=[
            pl.BlockSpec((NC, _BM, D), lambda i: (0, i, 0)),
            pl.BlockSpec((_BM, D), lambda i: (i, 0)),
            pl.BlockSpec((_BM, NW), lambda i: (i, 0)),
            pl.BlockSpec((D, D), lambda i: (0, 0)),
        ],
        out_specs=pl.BlockSpec((_BM, D), lambda i: (i, 0)),
        out_shape=jax.ShapeDtypeStruct((N, D), jnp.float32),
    )(p, g1, degp, W2)


def _tc3_body(p_ref, g_ref, degp_ref, w_ref, out_ref):
    dinv = _dinv_of(degp_ref[...])
    out2 = (p_ref[0] + p_ref[1] + g_ref[...]) * dinv
    logits = jnp.dot(out2, w_ref[...], preferred_element_type=jnp.float32)
    m = jnp.max(logits, axis=1, keepdims=True)
    e = jnp.exp(logits - m)
    out_ref[...] = e / jnp.sum(e, axis=1, keepdims=True)


def _tc3(p, g2, degp, Wlin):
    return pl.pallas_call(
        _tc3_body,
        grid=(N // _BM,),
        in_specs=[
            pl.BlockSpec((NC, _BM, D), lambda i: (0, i, 0)),
            pl.BlockSpec((_BM, D), lambda i: (i, 0)),
            pl.BlockSpec((_BM, NW), lambda i: (i, 0)),
            pl.BlockSpec((D, C), lambda i: (0, 0)),
        ],
        out_specs=pl.BlockSpec((_BM, C), lambda i: (i, 0)),
        out_shape=jax.ShapeDtypeStruct((N, C), jnp.float32),
    )(p, g2, degp, Wlin)


def kernel(x, edge_index, emb_table, W1, W2, Wlin):
    padlen = E_PAD - E
    spread = jnp.arange(padlen, dtype=jnp.int32) % 128
    pad2 = jnp.stack([spread, N + spread])
    eidx = jnp.concatenate([edge_index, pad2], axis=1).reshape(
        2 * PAD_BLOCKS, CROWS)

    padlen_x = EMB_NB * NW * 128 - N * BAG
    xpad = (jnp.arange(padlen_x, dtype=jnp.int32) % 128).reshape(-1, BAG)
    x_p = jnp.concatenate([x, xpad], axis=0).reshape(EMB_NB * NW, 128)

    h_sum, degp = _sc_emb_deg(x_p, eidx, emb_table)
    degp_t = jnp.swapaxes(degp, 0, 1)
    g1 = _tc1(h_sum, degp_t, W1)
    p1 = _sc_scatter(g1, eidx)
    g2 = _tc2(p1, g1, degp_t, W2)
    p2 = _sc_scatter(g2, eidx)
    return _tc3(p2, g2, degp_t, Wlin)


# speedup vs baseline: 1.1218x; 1.1218x over previous
import dataclasses
import functools

import jax
import jax.numpy as jnp
from jax import lax
from jax.experimental import pallas as pl
from jax.experimental.pallas import tpu as pltpu
from jax.experimental.pallas import tpu_sc as plsc

N = 10000
E = 320000
BAG = 16
D = 128
C = 16

NC = 2
NS = 16
NW = NC * NS
L = 16

EMB_BLOCKS = (N * BAG) // 128
EMB_NB = 40
EMB_ROWS = 10240
EDGE_BLOCKS = E // 128
ROW_BLOCKS = N // 8

CB = 160
CROWS = 64
PAD_BLOCKS = CB * NS * NC
E_PAD = PAD_BLOCKS * CROWS
NB = 80
ZB = 16
ACC_ROWS = 10240

_mesh = plsc.VectorSubcoreMesh(core_axis_name="c", subcore_axis_name="s")

_sc_params = pltpu.CompilerParams()
if "needs_layout_passes" in pltpu.CompilerParams.__dataclass_fields__:
    _sc_params = dataclasses.replace(_sc_params, needs_layout_passes=False)


def _ceil_div(a, b):
    return (a + b - 1) // b


@functools.partial(
    pl.kernel,
    out_type=(
        jax.ShapeDtypeStruct((EMB_ROWS, D), jnp.float32),
        jax.ShapeDtypeStruct((NW, N), jnp.float32),
    ),
    mesh=_mesh,
    scratch_types=[
        pltpu.VMEM((EMB_NB, 128), jnp.int32),
        pltpu.VMEM((CB // 2, CROWS), jnp.int32),
        pltpu.VMEM((128, D), jnp.float32),
        pltpu.VMEM((128, D), jnp.float32),
        pltpu.VMEM((8, D), jnp.float32),
        pltpu.VMEM((8, D), jnp.float32),
        pltpu.VMEM((N,), jnp.float32),
        pltpu.SemaphoreType.DMA,
        pltpu.SemaphoreType.DMA,
        pltpu.SemaphoreType.DMA,
        pltpu.SemaphoreType.DMA,
    ],
    compiler_params=_sc_params,
)
def _sc_emb_deg(x_hbm, eidx_hbm, table_hbm, hsum_hbm, degp_hbm,
                idx_v, didx_v, rows0, rows1, acc0, acc1, hist_v,
                gsem0, gsem1, wsem0, wsem1):
    c = lax.axis_index("c")
    s = lax.axis_index("s")
    wid = s * NC + c

    ones = jnp.ones((L,), jnp.float32)
    zeros = jnp.zeros((L,), jnp.float32)

    @pl.loop(0, N // L)
    def _(k):
        hist_v[pl.ds(k * L, L)] = zeros

    lo_e = PAD_BLOCKS + wid * CB
    for h in range(2):
        pltpu.sync_copy(eidx_hbm.at[pl.ds(lo_e + h * (CB // 2), CB // 2)],
                        didx_v)

        @pl.loop(0, CB // 2)
        def _(b):
            for j in range(CROWS // L):
                idx = didx_v.at[pl.ds(b, 1)][0, pl.ds(j * L, L)]
                plsc.addupdate_scatter(hist_v, [idx], ones, mask=idx < N)

    pltpu.sync_copy(hist_v, degp_hbm.at[wid])

    lo_x = wid * EMB_NB
    pltpu.sync_copy(x_hbm.at[pl.ds(lo_x, EMB_NB)], idx_v.at[pl.ds(0, EMB_NB)])

    def gather(blk, buf, sem):
        pltpu.async_copy(table_hbm.at[idx_v.at[blk]], buf, sem)

    def wait_gather(buf, sem):
        pltpu.make_async_copy(table_hbm.at[idx_v.at[0]], buf, sem).wait()

    def wait_write(buf, sem):
        pltpu.make_async_copy(buf, hsum_hbm.at[pl.ds(0, 8)], sem).wait()

    def reduce_block(rbuf, abuf):
        @pl.loop(0, 8)
        def _(n):
            for j in range(D // L):
                sl = pl.ds(j * L, L)
                v = rbuf.at[pl.ds(n * BAG, 1)][0, sl]
                for t in range(1, BAG):
                    v = v + rbuf.at[pl.ds(n * BAG + t, 1)][0, sl]
                abuf.at[pl.ds(n, 1)][0, sl] = v

    gather(0, rows0, gsem0)

    @pl.loop(0, EMB_NB // 2)
    def _(i):
        e = 2 * i
        o = e + 1
        wait_gather(rows0, gsem0)
        gather(o, rows1, gsem1)

        @pl.when(i > 0)
        def _():
            wait_write(acc0, wsem0)

        reduce_block(rows0, acc0)
        pltpu.async_copy(acc0, hsum_hbm.at[pl.ds((lo_x + e) * 8, 8)], wsem0)
        wait_gather(rows1, gsem1)

        @pl.when(i < EMB_NB // 2 - 1)
        def _():
            gather(e + 2, rows0, gsem0)

        @pl.when(i > 0)
        def _():
            wait_write(acc1, wsem1)

        reduce_block(rows1, acc1)
        pltpu.async_copy(acc1, hsum_hbm.at[pl.ds((lo_x + o) * 8, 8)], wsem1)

    wait_write(acc0, wsem0)
    wait_write(acc1, wsem1)


@functools.partial(
    pl.kernel,
    out_type=jax.ShapeDtypeStruct((NC, ACC_ROWS, D), jnp.float32),
    mesh=_mesh,
    scratch_types=[
        pltpu.VMEM((CB // 4, CROWS), jnp.int32),
        pltpu.VMEM((CB // 4, CROWS), jnp.int32),
        pltpu.VMEM((4, CROWS, D), jnp.float32),
        pltpu.VMEM((ZB, D), jnp.float32),
        pltpu.VMEM_SHARED((ACC_ROWS, D), jnp.float32),
        [pltpu.SemaphoreType.DMA] * 4,
        [pltpu.SemaphoreType.DMA] * 4,
        pltpu.SemaphoreType.DMA,
    ],
    compiler_params=_sc_params,
)
def _sc_scatter(g_hbm, eidx_hbm, out_hbm,
                sidx_v, didx_v, rows_v, zero_v, acc_sh,
                gsems, ssems, osem):
    c = lax.axis_index("c")
    s = lax.axis_index("s")

    zeros = jnp.zeros((L,), jnp.float32)

    @pl.loop(0, ZB)
    def _(r):
        for j in range(D // L):
            zero_v.at[pl.ds(r, 1)][0, pl.ds(j * L, L)] = zeros

    cpt = ACC_ROWS // ZB // NS
    for i in range(cpt):
        b = s * cpt + i
        pltpu.async_copy(zero_v, acc_sh.at[pl.ds(b * ZB, ZB)], osem)
    for i in range(cpt):
        pltpu.make_async_copy(zero_v, acc_sh.at[pl.ds(0, ZB)], osem).wait()

    plsc.subcore_barrier()

    bufs = [rows_v.at[k] for k in range(4)]

    def gather(blk, k):
        pltpu.async_copy(g_hbm.at[sidx_v.at[blk]], bufs[k], gsems[k])

    def wait_gather(k):
        pltpu.make_async_copy(g_hbm.at[sidx_v.at[0]], bufs[k],
                              gsems[k]).wait()

    def scatter(blk, k):
        pltpu.async_copy(bufs[k], acc_sh.at[didx_v.at[blk]], ssems[k],
                         add=True)

    def wait_scatter(k):
        pltpu.make_async_copy(bufs[k], acc_sh.at[didx_v.at[0]],
                              ssems[k]).wait()

    CB2 = CB // 4
    lo = c * (CB * NS) + s * CB
    for h in range(4):
        pltpu.sync_copy(eidx_hbm.at[pl.ds(lo + h * CB2, CB2)], sidx_v)
        pltpu.sync_copy(eidx_hbm.at[pl.ds(PAD_BLOCKS + lo + h * CB2, CB2)],
                        didx_v)
        gather(0, 0)
        gather(2, 2)

        @pl.loop(0, CB2 // 4)
        def _(i):
            a = 4 * i
            wait_gather(0)
            scatter(a, 0)

            @pl.when(i > 0)
            def _():
                wait_scatter(1)

            gather(a + 1, 1)
            wait_gather(2)
            scatter(a + 2, 2)

            @pl.when(i > 0)
            def _():
                wait_scatter(3)

            gather(a + 3, 3)
            wait_gather(1)
            scatter(a + 1, 1)

            @pl.when(i < CB2 // 4 - 1)
            def _():
                wait_scatter(0)
                gather(a + 4, 0)

            wait_gather(3)
            scatter(a + 3, 3)

            @pl.when(i < CB2 // 4 - 1)
            def _():
                wait_scatter(2)
                gather(a + 6, 2)

        for k in range(4):
            wait_scatter(k)

    plsc.subcore_barrier()

    for i in range(cpt):
        b = s * cpt + i
        pltpu.async_copy(acc_sh.at[pl.ds(b * ZB, ZB)],
                         out_hbm.at[c, pl.ds(b * ZB, ZB)], osem)
    for i in range(cpt):
        pltpu.make_async_copy(acc_sh.at[pl.ds(0, ZB)],
                              out_hbm.at[c, pl.ds(0, ZB)], osem).wait()


_BM = 2000


def _dinv_of(degp_block):
    deg = jnp.sum(degp_block, axis=1, keepdims=True) + 1.0
    return lax.rsqrt(deg)


def _tc1_body(hs_ref, degp_ref, w_ref, g_ref):
    h = jnp.maximum(hs_ref[...] * (1.0 / BAG), 0.0)
    hw = jnp.dot(h, w_ref[...], preferred_element_type=jnp.float32)
    g_ref[...] = hw * _dinv_of(degp_ref[...])


def _tc1(h_sum, degp, W1):
    return pl.pallas_call(
        _tc1_body,
        grid=(N // _BM,),
        in_specs=[
            pl.BlockSpec((_BM, D), lambda i: (i, 0)),
            pl.BlockSpec((_BM, NW), lambda i: (i, 0)),
            pl.BlockSpec((D, D), lambda i: (0, 0)),
        ],
        out_specs=pl.BlockSpec((_BM, D), lambda i: (i, 0)),
        out_shape=jax.ShapeDtypeStruct((N, D), jnp.float32),
    )(h_sum, degp, W1)


def _tc2_body(p_ref, g_ref, degp_ref, w_ref, out_ref):
    dinv = _dinv_of(degp_ref[...])
    ssum = p_ref[0] + p_ref[1] + g_ref[...]
    out1 = jnp.maximum(ssum * dinv, 0.0)
    hw = jnp.dot(out1, w_ref[...], preferred_element_type=jnp.float32)
    out_ref[...] = hw * dinv


def _tc2(p, g1, degp, W2):
    return pl.pallas_call(
        _tc2_body,
        grid=(N // _BM,),
        in_specs=[
            pl.BlockSpec((NC, _BM, D), lambda i: (0, i, 0)),
            pl.BlockSpec((_BM, D), lambda i: (i, 0)),
            pl.BlockSpec((_BM, NW), lambda i: (i, 0)),
            pl.BlockSpec((D, D), lambda i: (0, 0)),
        ],
        out_specs=pl.BlockSpec((_BM, D), lambda i: (i, 0)),
        out_shape=jax.ShapeDtypeStruct((N, D), jnp.float32),
    )(p, g1, degp, W2)


def _tc3_body(p_ref, g_ref, degp_ref, w_ref, out_ref):
    dinv = _dinv_of(degp_ref[...])
    out2 = (p_ref[0] + p_ref[1] + g_ref[...]) * dinv
    logits = jnp.dot(out2, w_ref[...], preferred_element_type=jnp.float32)
    m = jnp.max(logits, axis=1, keepdims=True)
    e = jnp.exp(logits - m)
    out_ref[...] = e / jnp.sum(e, axis=1, keepdims=True)


def _tc3(p, g2, degp, Wlin):
    return pl.pallas_call(
        _tc3_body,
        grid=(N // _BM,),
        in_specs=[
            pl.BlockSpec((NC, _BM, D), lambda i: (0, i, 0)),
            pl.BlockSpec((_BM, D), lambda i: (i, 0)),
            pl.BlockSpec((_BM, NW), lambda i: (i, 0)),
            pl.BlockSpec((D, C), lambda i: (0, 0)),
        ],
        out_specs=pl.BlockSpec((_BM, C), lambda i: (i, 0)),
        out_shape=jax.ShapeDtypeStruct((N, C), jnp.float32),
    )(p, g2, degp, Wlin)


def kernel(x, edge_index, emb_table, W1, W2, Wlin):
    padlen = E_PAD - E
    spread = jnp.arange(padlen, dtype=jnp.int32) % 128
    pad2 = jnp.stack([spread, N + spread])
    eidx = jnp.concatenate([edge_index, pad2], axis=1).reshape(
        2 * PAD_BLOCKS, CROWS)

    padlen_x = EMB_NB * NW * 128 - N * BAG
    xpad = (jnp.arange(padlen_x, dtype=jnp.int32) % 128).reshape(-1, BAG)
    x_p = jnp.concatenate([x, xpad], axis=0).reshape(EMB_NB * NW, 128)

    h_sum, degp = _sc_emb_deg(x_p, eidx, emb_table)
    degp_t = jnp.swapaxes(degp, 0, 1)
    g1 = _tc1(h_sum, degp_t, W1)
    p1 = _sc_scatter(g1, eidx)
    g2 = _tc2(p1, g1, degp_t, W2)
    p2 = _sc_scatter(g2, eidx)
    return _tc3(p2, g2, degp_t, Wlin)
